# Initial kernel scaffold; baseline (speedup 1.0000x reference)
import jax
import jax.numpy as jnp
from jax.experimental import pallas as pl


def kernel(x, adj, edges, z_table, W1, b1, W2, b2, W3, b3):
    raise NotImplementedError("write your pallas kernel here")


# speedup vs baseline: 1.6919x; 1.6919x over previous
import functools
import jax
import jax.numpy as jnp
from jax import lax
from jax.experimental import pallas as pl
from jax.experimental.pallas import tpu as pltpu
from jax.experimental.pallas import tpu_sc as plsc

N = 4096
E = 4096
H = 128


_BM = 1024
_BN = 1024
_BK = 512


def _a2_body(a_ik, b_kj, a_ij, a2_out, acc):
    i = pl.program_id(0)
    j = pl.program_id(1)
    k = pl.program_id(2)
    nk = pl.num_programs(2)

    @pl.when(k == 0)
    def _init():
        acc[:] = jnp.zeros_like(acc)

    acc[:] += jnp.dot(a_ik[:], b_kj[:], preferred_element_type=jnp.float32)

    @pl.when(k == nk - 1)
    def _fin():
        a1blk = a_ij[:].astype(jnp.float32)
        rows = i * _BM + lax.broadcasted_iota(jnp.int32, (_BM, _BN), 0)
        cols = j * _BN + lax.broadcasted_iota(jnp.int32, (_BM, _BN), 1)
        off_diag = rows != cols
        a2_out[:] = jnp.where((acc[:] > 0.0) & (a1blk == 0.0) & off_diag,
                              1.0, 0.0)


def _compute_a2(a1_bf):
    grid = (N // _BM, N // _BN, N // _BK)
    return pl.pallas_call(
        _a2_body,
        grid=grid,
        in_specs=[
            pl.BlockSpec((_BM, _BK), lambda i, j, k: (i, k)),
            pl.BlockSpec((_BK, _BN), lambda i, j, k: (k, j)),
            pl.BlockSpec((_BM, _BN), lambda i, j, k: (i, j)),
        ],
        out_specs=pl.BlockSpec((_BM, _BN), lambda i, j, k: (i, j)),
        out_shape=jax.ShapeDtypeStruct((N, N), jnp.float32),
        scratch_shapes=[pltpu.VMEM((_BM, _BN), jnp.float32)],
    )(a1_bf, a1_bf, a1_bf)


_BE = 1024

_PAIRS = ((0, 0), (1, 1), (1, 2), (2, 1), (1, 3), (3, 1), (2, 2), (2, 3),
          (3, 2))


def _mlp_body(cnt, z, w1, b1, w2, b2, w3, b3, out):
    zt = z[:]
    t = jnp.stack([zt[a] + zt[b] for a, b in _PAIRS]
                  + [jnp.zeros((H,), jnp.float32)] * 7)
    cblk = cnt[:]
    pooled_t = jnp.dot(t.T, cblk, preferred_element_type=jnp.float32,
                       precision=lax.Precision.HIGHEST)
    total = cblk[9:10, :]
    out_t = pooled_t / total
    h1 = jnp.maximum(jnp.dot(w1[:].T, out_t,
                             preferred_element_type=jnp.float32,
                             precision=lax.Precision.HIGHEST)
                     + b1[:][:, None], 0.0)
    h2 = jnp.maximum(jnp.dot(w2[:].T, h1,
                             preferred_element_type=jnp.float32,
                             precision=lax.Precision.HIGHEST)
                     + b2[:][:, None], 0.0)
    logit = jnp.dot(w3[:].T, h2,
                    preferred_element_type=jnp.float32,
                    precision=lax.Precision.HIGHEST) + b3[0]
    out[:] = jnp.broadcast_to(logit, (8, _BE))


def _mlp_head(counts, z_table, W1, b1, W2, b2, W3, b3):
    grid = (E // _BE,)
    logit_t = pl.pallas_call(
        _mlp_body,
        grid=grid,
        in_specs=[
            pl.BlockSpec((16, _BE), lambda e: (0, e)),
            pl.BlockSpec((4, H), lambda e: (0, 0)),
            pl.BlockSpec((H, H), lambda e: (0, 0)),
            pl.BlockSpec((H,), lambda e: (0,)),
            pl.BlockSpec((H, H), lambda e: (0, 0)),
            pl.BlockSpec((H,), lambda e: (0,)),
            pl.BlockSpec((H, 1), lambda e: (0, 0)),
            pl.BlockSpec(memory_space=pltpu.SMEM),
        ],
        out_specs=pl.BlockSpec((8, _BE), lambda e: (0, e)),
        out_shape=jax.ShapeDtypeStruct((8, E), jnp.float32),
    )(counts, z_table, W1, b1, W2, b2, W3, b3)
    return logit_t[0].reshape(E, 1)


def _counts_middle(adj, a2, u, v):
    g1u, g1v, g2u, g2v = adj[u], adj[v], a2[u], a2[v]
    c11 = (g1u * g1v).sum(1)
    c12 = (g1u * g2v).sum(1)
    c21 = (g2u * g1v).sum(1)
    c22 = (g2u * g2v).sum(1)
    d1u, d1v = g1u.sum(1), g1v.sum(1)
    d2u, d2v = g2u.sum(1), g2v.sum(1)
    a1uv = adj[u, v]
    a2uv = a2[u, v]
    c1i = d1u - c11 - c12 - a1uv
    ci1 = d1v - c11 - c21 - a1uv
    c2i = d2u - c21 - c22 - a2uv
    ci2 = d2v - c12 - c22 - a2uv
    c00 = jnp.full_like(c11, 2.0)
    total = c00 + c11 + c12 + c21 + c1i + ci1 + c22 + c2i + ci2
    zero = jnp.zeros_like(c11)
    return jnp.stack([c00, c11, c12, c21, c1i, ci1, c22, c2i, ci2, total,
                      zero, zero, zero, zero, zero, zero])


def kernel(x, adj, edges, z_table, W1, b1, W2, b2, W3, b3):
    del x
    a1_bf = adj.astype(jnp.bfloat16)
    a2 = _compute_a2(a1_bf)
    u = edges[0]
    v = edges[1]
    counts = _counts_middle(adj, a2, u, v)
    return _mlp_head(counts, z_table, W1, b1, W2, b2, W3, b3)
